# Initial kernel scaffold; baseline (speedup 1.0000x reference)
import jax
import jax.numpy as jnp
from jax.experimental import pallas as pl


def kernel(edge_index, edge_feat, node_feat, state_feat, params):
    raise NotImplementedError("write your pallas kernel here")


# speedup vs baseline: 2.2696x; 2.2696x over previous
import functools
import jax
import jax.numpy as jnp
from jax import lax
from jax.experimental import pallas as pl
from jax.experimental.pallas import tpu as pltpu
from jax.experimental.pallas import tpu_sc as plsc

_NN = 50000
_NG = 25
_NPG = 2000
_NE = 800000
_EPG = 32000
_LOG2 = 0.6931471805599453

_NW = 32
_BPW = _NE // _NW
_GC = 128
_NCH = (_BPW + _GC - 1) // _GC
_TAIL = _BPW - _GC

_EC = 4000
_ECH = _NE // _EC
_CPG = _EPG // _EC

_SC_CH = 640
_SC_NCH = _EPG // _SC_CH
_SC_GRP = _SC_CH // 16


def _sp2(x):
    return jax.nn.softplus(x) - _LOG2


def _sc_gather(table, idx_s, idx_d):
    mesh = plsc.VectorSubcoreMesh(core_axis_name="c", subcore_axis_name="s")

    @functools.partial(
        pl.kernel, mesh=mesh,
        compiler_params=pltpu.CompilerParams(use_tc_tiling_on_sc=False, needs_layout_passes=False),
        out_type=(jax.ShapeDtypeStruct((_NE, 32), jnp.float32),
                  jax.ShapeDtypeStruct((_NE, 32), jnp.float32)),
        scratch_types=[
            pltpu.VMEM((_NCH, _GC), jnp.int32),
            pltpu.VMEM((_NCH, _GC), jnp.int32),
            pltpu.VMEM((_GC, 32), jnp.float32),
            pltpu.VMEM((_GC, 32), jnp.float32),
            pltpu.SemaphoreType.DMA,
            pltpu.SemaphoreType.DMA,
        ],
    )
    def k(table_hbm, is_hbm, id_hbm, o1_hbm, o2_hbm, iv1, iv2, r1, r2, s1, s2):
        wid = lax.axis_index("s") * 2 + lax.axis_index("c")
        base = wid * _BPW
        pltpu.sync_copy(is_hbm.at[wid], iv1)
        pltpu.sync_copy(id_hbm.at[wid], iv2)

        def body(i, carry):
            off = base + jnp.minimum(i * _GC, _TAIL)
            a = pltpu.async_copy(table_hbm.at[iv1.at[i]], r1, s1)
            b = pltpu.async_copy(table_hbm.at[iv2.at[i]], r2, s2)
            a.wait()
            b.wait()
            pltpu.sync_copy(r1, o1_hbm.at[pl.ds(off, _GC)])
            pltpu.sync_copy(r2, o2_hbm.at[pl.ds(off, _GC)])
            return carry

        lax.fori_loop(0, _NCH, body, 0)

    return k(table, idx_s, idx_d)


def _sc_scatter(enew, dst, zrows, zcnt):
    mesh = plsc.VectorSubcoreMesh(core_axis_name="c", subcore_axis_name="s")

    @functools.partial(
        pl.kernel, mesh=mesh,
        compiler_params=pltpu.CompilerParams(use_tc_tiling_on_sc=False, needs_layout_passes=False),
        out_type=(jax.ShapeDtypeStruct((_NN, 32), jnp.float32),
                  jax.ShapeDtypeStruct((_NN, 16), jnp.float32)),
        scratch_types=[
            pltpu.VMEM((_SC_CH, 32), jnp.float32),
            pltpu.VMEM((_SC_CH,), jnp.int32),
            pltpu.VMEM((_NPG, 32), jnp.float32),
            pltpu.VMEM((_NPG, 16), jnp.float32),
        ],
    )
    def k(e_hbm, d_hbm, zr_hbm, zc_hbm, es_hbm, cn_hbm, ebuf, dbuf, acc, cnt):
        wid = lax.axis_index("s") * 2 + lax.axis_index("c")

        @pl.when(wid < _NG)
        def _():
            pltpu.sync_copy(zr_hbm, acc)
            pltpu.sync_copy(zc_hbm, cnt)
            ji = lax.iota(jnp.int32, 16)
            ones = jnp.ones((16,), jnp.float32)
            nbase = wid * _NPG
            ebase = wid * _EPG

            def chunk(i, carry):
                eb = ebase + i * _SC_CH
                pltpu.sync_copy(e_hbm.at[pl.ds(eb, _SC_CH)], ebuf)
                pltpu.sync_copy(d_hbm.at[pl.ds(eb, _SC_CH)], dbuf)

                def grp(j, c2):
                    dv = dbuf[pl.ds(j * 16, 16)]
                    loc = dv - nbase
                    rows = j * 16 + ji
                    plsc.addupdate_scatter(cnt, [loc, ji], ones)
                    for c in range(32):
                        cf = jnp.full((16,), c, jnp.int32)
                        vals = plsc.load_gather(ebuf, [rows, cf])
                        plsc.addupdate_scatter(acc, [loc, cf], vals)
                    return c2

                lax.fori_loop(0, _SC_GRP, grp, 0)
                return carry

            lax.fori_loop(0, _SC_NCH, chunk, 0)
            pltpu.sync_copy(acc, es_hbm.at[pl.ds(nbase, _NPG)])
            pltpu.sync_copy(cnt, cn_hbm.at[pl.ds(nbase, _NPG)])

    return k(enew, dst, zrows, zcnt)


def _dot(a, b):
    return jnp.dot(a, b, preferred_element_type=jnp.float32)


def _enc_edge_body(x_ref, w1, b1, w2, b2, o_ref):
    x = x_ref[...]
    g = jnp.exp(-(x * x) * 4.0)
    h = _sp2(g * w1[...] + b1[...])
    o_ref[...] = _sp2(_dot(h, w2[...]) + b2[...])


def _enc_edge(x, ps):
    (w1, b1), (w2, b2) = ps
    full = lambda s: pl.BlockSpec(s, lambda i: (0, 0))
    return pl.pallas_call(
        _enc_edge_body,
        grid=(_ECH,),
        in_specs=[pl.BlockSpec((_EC, 1), lambda i: (i, 0)),
                  full((1, 64)), full((1, 64)), full((64, 32)), full((1, 32))],
        out_specs=pl.BlockSpec((_EC, 32), lambda i: (i, 0)),
        out_shape=jax.ShapeDtypeStruct((_NE, 32), jnp.float32),
    )(x, w1.reshape(1, 64), b1.reshape(1, 64), w2, b2.reshape(1, 32))


def _enc_node_body(v_ref, emb, w1, b1, w2, b2, o_ref, f_ref):
    v = v_ref[...]
    ci = lax.broadcasted_iota(jnp.int32, (_NPG, 90), 1)
    sent = jnp.where((ci < 89) & (v > 0), ci, 2000)
    ntype = jnp.min(sent, axis=1, keepdims=True)
    ntype = jnp.where(ntype == 2000, 0, ntype)
    oh = (ntype == lax.broadcasted_iota(jnp.int32, (_NPG, 96), 1)).astype(jnp.float32)
    nf16 = _dot(oh, emb[...])
    h = _sp2(_dot(nf16, w1[...]) + b1[...])
    o_ref[...] = _sp2(_dot(h, w2[...]) + b2[...])
    flagv = jnp.sum(jnp.where(ci == 89, v, 0), axis=1, keepdims=True)
    li = lax.broadcasted_iota(jnp.int32, (_NPG, 1), 0)
    idxv = jnp.where(flagv != 0, li, 2000)
    fv = jnp.minimum(jnp.min(idxv), 20)
    f_ref[...] = jnp.full((1, 1, 128), fv.astype(jnp.float32))


def _enc_node(v, emb, ps):
    (w1, b1), (w2, b2) = ps
    embp = jnp.zeros((96, 16), jnp.float32).at[:89].set(emb)
    full = lambda s: pl.BlockSpec(s, lambda i: (0, 0))
    return pl.pallas_call(
        _enc_node_body,
        grid=(_NG,),
        in_specs=[pl.BlockSpec((_NPG, 90), lambda i: (i, 0)),
                  full((96, 16)), full((16, 64)), full((1, 64)),
                  full((64, 32)), full((1, 32))],
        out_specs=[pl.BlockSpec((_NPG, 32), lambda i: (i, 0)),
                   pl.BlockSpec((1, 1, 128), lambda i: (i, 0, 0))],
        out_shape=[jax.ShapeDtypeStruct((_NN, 32), jnp.float32),
                   jax.ShapeDtypeStruct((_NG, 1, 128), jnp.float32)],
    )(v, embp, w1, b1.reshape(1, 64), w2, b2.reshape(1, 32))


def _enc_state_body(sfeat, focus, semb, w1, b1, w2, b2, o_ref):
    fv = focus[...][:, 0:1].astype(jnp.int32)
    oh = (fv == lax.broadcasted_iota(jnp.int32, (32, 24), 1)).astype(jnp.float32)
    ff = _dot(oh, semb[...])
    x = jnp.concatenate([sfeat[...], ff], axis=1)
    h = _sp2(_dot(x, w1[...]) + b1[...])
    o_ref[...] = _sp2(_dot(h, w2[...]) + b2[...])


def _enc_state(sfeat, focus, semb, ps):
    (w1, b1), (w2, b2) = ps
    sembp = jnp.zeros((24, 8), jnp.float32).at[:21].set(semb)
    sfp = jnp.zeros((32, 8), jnp.float32).at[:_NG].set(sfeat)
    fp = jnp.zeros((32, 128), jnp.float32).at[:_NG].set(focus)
    return pl.pallas_call(
        _enc_state_body,
        out_shape=jax.ShapeDtypeStruct((32, 32), jnp.float32),
    )(sfp, fp, sembp, w1, b1.reshape(1, 64), w2, b2.reshape(1, 32))


def _dense32_body(x_ref, w, b, o_ref):
    o_ref[...] = _sp2(_dot(x_ref[...], w[...]) + b[...])


def _dense_table(x, ps):
    (w, b), = ps
    full = lambda s: pl.BlockSpec(s, lambda i: (0, 0))
    return pl.pallas_call(
        _dense32_body,
        grid=(_NG,),
        in_specs=[pl.BlockSpec((_NPG, 32), lambda i: (i, 0)),
                  full((32, 32)), full((1, 32))],
        out_specs=pl.BlockSpec((_NPG, 32), lambda i: (i, 0)),
        out_shape=jax.ShapeDtypeStruct((_NN, 32), jnp.float32),
    )(x, w, b.reshape(1, 32))


def _dense_state(x, ps):
    (w, b), = ps
    return pl.pallas_call(
        _dense32_body,
        out_shape=jax.ShapeDtypeStruct((32, 32), jnp.float32),
    )(x, w, b.reshape(1, 32))


def _onehot_row(ref, g, rows):
    sel = (lax.broadcasted_iota(jnp.int32, (rows, 1), 0) == g).astype(jnp.float32)
    return _dot(sel.reshape(1, rows), ref[...])


def _edge_block_body(has_dense):
    def body(ep_ref, gs_ref, gd_ref, sd_ref,
             wd, bd, w1s, w1d, w1e, w1u, b1, w2, b2, w3, b3,
             en_ref, eo_ref, ps_ref):
        g = pl.program_id(0) // _CPG
        sg = _onehot_row(sd_ref, g, 32)
        ep = ep_ref[...]
        ed = _sp2(_dot(ep, wd[...]) + bd[...]) if has_dense else ep
        h1 = _sp2(_dot(gs_ref[...], w1s[...]) + _dot(gd_ref[...], w1d[...])
                  + _dot(ed, w1e[...]) + _dot(sg, w1u[...]) + b1[...])
        h2 = _sp2(_dot(h1, w2[...]) + b2[...])
        en = _sp2(_dot(h2, w3[...]) + b3[...])
        eo = en + ep
        en_ref[...] = en
        eo_ref[...] = eo
        s1 = jnp.sum(en, axis=0, keepdims=True)
        s2 = jnp.sum(eo, axis=0, keepdims=True)
        ps_ref[...] = jnp.concatenate(
            [s1, s2, jnp.zeros((1, 64), jnp.float32)], axis=1).reshape(1, 1, 128)
    return body


def _edge_block(ep, gs, gd, sd, bp, has_dense):
    (w1, b1), (w2, b2), (w3, b3) = bp['edge_func']
    if has_dense:
        (wd, bd), = bp['edge_dense']
    else:
        wd = jnp.zeros((32, 32), jnp.float32)
        bd = jnp.zeros((32,), jnp.float32)
    full = lambda s: pl.BlockSpec(s, lambda i: (0, 0))
    eb = lambda: pl.BlockSpec((_EC, 32), lambda i: (i, 0))
    return pl.pallas_call(
        _edge_block_body(has_dense),
        grid=(_ECH,),
        in_specs=[eb(), eb(), eb(), full((32, 32)),
                  full((32, 32)), full((1, 32)),
                  full((32, 64)), full((32, 64)), full((32, 64)), full((32, 64)),
                  full((1, 64)), full((64, 64)), full((1, 64)),
                  full((64, 32)), full((1, 32))],
        out_specs=[eb(), eb(), pl.BlockSpec((1, 1, 128), lambda i: (i, 0, 0))],
        out_shape=[jax.ShapeDtypeStruct((_NE, 32), jnp.float32),
                   jax.ShapeDtypeStruct((_NE, 32), jnp.float32),
                   jax.ShapeDtypeStruct((_ECH, 1, 128), jnp.float32)],
    )(ep, gs, gd, sd, wd, bd.reshape(1, 32),
      w1[0:32], w1[32:64], w1[64:96], w1[96:128], b1.reshape(1, 64),
      w2, b2.reshape(1, 64), w3, b3.reshape(1, 32))


def _node_block_body(np_ref, tb_ref, es_ref, ct_ref, sd_ref,
                     w1n, w1v, w1u, b1, w2, b2, w3, b3,
                     no_ref, ps_ref):
    g = pl.program_id(0)
    sg = _onehot_row(sd_ref, g, 32)
    ve = es_ref[...] / jnp.maximum(jnp.sum(ct_ref[...], axis=1, keepdims=True), 1.0)
    h1 = _sp2(_dot(tb_ref[...], w1n[...]) + _dot(ve, w1v[...])
              + _dot(sg, w1u[...]) + b1[...])
    h2 = _sp2(_dot(h1, w2[...]) + b2[...])
    nn = _sp2(_dot(h2, w3[...]) + b3[...])
    no = nn + np_ref[...]
    no_ref[...] = no
    s1 = jnp.sum(nn, axis=0, keepdims=True)
    s2 = jnp.sum(no, axis=0, keepdims=True)
    ps_ref[...] = jnp.concatenate(
        [s1, s2, jnp.zeros((1, 64), jnp.float32)], axis=1).reshape(1, 1, 128)


def _node_block(nprev, table, esum, cnt, sd, bp):
    (w1, b1), (w2, b2), (w3, b3) = bp['node_func']
    full = lambda s: pl.BlockSpec(s, lambda i: (0, 0))
    nb = lambda: pl.BlockSpec((_NPG, 32), lambda i: (i, 0))
    return pl.pallas_call(
        _node_block_body,
        grid=(_NG,),
        in_specs=[nb(), nb(), nb(), pl.BlockSpec((_NPG, 16), lambda i: (i, 0)),
                  full((32, 32)),
                  full((32, 64)), full((32, 64)), full((32, 64)), full((1, 64)),
                  full((64, 64)), full((1, 64)), full((64, 32)), full((1, 32))],
        out_specs=[nb(), pl.BlockSpec((1, 1, 128), lambda i: (i, 0, 0))],
        out_shape=[jax.ShapeDtypeStruct((_NN, 32), jnp.float32),
                   jax.ShapeDtypeStruct((_NG, 1, 128), jnp.float32)],
    )(nprev, table, esum, cnt, sd,
      w1[0:32], w1[32:64], w1[64:96], b1.reshape(1, 64),
      w2, b2.reshape(1, 64), w3, b3.reshape(1, 32))


def _state_func_body(sp_ref, sd_ref, ue_ref, uv_ref, w1s, w1e, w1v, b1, w2, b2, w3, b3, o_ref):
    h1 = _sp2(_dot(sd_ref[...], w1s[...]) + _dot(ue_ref[...], w1e[...])
              + _dot(uv_ref[...], w1v[...]) + b1[...])
    h2 = _sp2(_dot(h1, w2[...]) + b2[...])
    o_ref[...] = _sp2(_dot(h2, w3[...]) + b3[...]) + sp_ref[...]


def _state_func(sprev, sd, ue, uv, bp):
    (w1, b1), (w2, b2), (w3, b3) = bp['state_func']
    return pl.pallas_call(
        _state_func_body,
        out_shape=jax.ShapeDtypeStruct((32, 32), jnp.float32),
    )(sprev, sd, ue, uv,
      w1[0:32], w1[32:64], w1[64:96], b1.reshape(1, 64),
      w2, b2.reshape(1, 64), w3, b3.reshape(1, 32))


def _final_body(nm_ref, em_ref, s3_ref, w1, b1, w2, b2, w3, b3, o_ref):
    z = jnp.zeros((32, 32), jnp.float32)
    x = jnp.concatenate([z, nm_ref[...], z, em_ref[...], s3_ref[...],
                         jnp.zeros((32, 96), jnp.float32)], axis=1)
    h1 = _sp2(_dot(x, w1[...]) + b1[...])
    h2 = _sp2(_dot(h1, w2[...]) + b2[...])
    o_ref[...] = _dot(h2, w3[...]) + b3[...]


def _final(nmean, emean, s3, ps):
    (w1, b1), (w2, b2), (w3, b3) = ps
    w1p = jnp.zeros((256, 32), jnp.float32).at[:160].set(w1)
    return pl.pallas_call(
        _final_body,
        out_shape=jax.ShapeDtypeStruct((32, 88), jnp.float32),
    )(nmean, emean, s3, w1p, b1.reshape(1, 32),
      w2, b2.reshape(1, 16), w3, b3.reshape(1, 88))


def _mk_gidx(idx):
    x = idx.reshape(_NW, _BPW)
    main = x[:, :_GC * (_NCH - 1)].reshape(_NW, _NCH - 1, _GC)
    tail = x[:, _TAIL:][:, None, :]
    return jnp.concatenate([main, tail], axis=1)


def kernel(edge_index, edge_feat, node_feat, state_feat, params):
    p = params
    src = edge_index[0].astype(jnp.int32)
    dst = edge_index[1].astype(jnp.int32)
    nfi = node_feat.astype(jnp.int32)

    ef = _enc_edge(edge_feat.reshape(_NE, 1), p['edge_enc'])
    nf, focus = _enc_node(nfi, p['node_embed'], p['node_enc'])
    sf = _enc_state(state_feat, focus[:, 0, :], p['state_embed'], p['state_enc'])

    gidx_s = _mk_gidx(src)
    gidx_d = _mk_gidx(dst)
    zrows = jnp.zeros((_NPG, 32), jnp.float32)
    zcnt = jnp.zeros((_NPG, 16), jnp.float32)

    epart3 = None
    npart3 = None
    for b, bp in enumerate(p['blocks']):
        if b > 0:
            table = _dense_table(nf, bp['node_dense'])
            sd = _dense_state(sf, bp['state_dense'])
        else:
            table, sd = nf, sf
        gs, gd = _sc_gather(table, gidx_s, gidx_d)
        enew, eout, epart = _edge_block(ef, gs, gd, sd, bp, has_dense=b > 0)
        esum, cnt = _sc_scatter(enew, dst, zrows, zcnt)
        nout, npart = _node_block(nf, table, esum, cnt, sd, bp)
        eg = epart.reshape(_NG, _CPG, 128).sum(axis=1)
        ng = npart[:, 0, :]
        ue = jnp.zeros((32, 32), jnp.float32).at[:_NG].set(eg[:, 0:32] / _EPG)
        uv = jnp.zeros((32, 32), jnp.float32).at[:_NG].set(ng[:, 0:32] / _NPG)
        sf = _state_func(sf, sd, ue, uv, bp)
        ef, nf = eout, nout
        epart3, npart3 = eg, ng

    nmean = jnp.zeros((32, 32), jnp.float32).at[:_NG].set(npart3[:, 32:64] / _NPG)
    emean = jnp.zeros((32, 32), jnp.float32).at[:_NG].set(epart3[:, 32:64] / _EPG)
    out = _final(nmean, emean, sf, p['out_proj'])
    return out[:_NG]


# speedup vs baseline: 2.2989x; 1.0129x over previous
import functools
import jax
import jax.numpy as jnp
from jax import lax
from jax.experimental import pallas as pl
from jax.experimental.pallas import tpu as pltpu
from jax.experimental.pallas import tpu_sc as plsc

_NN = 50000
_NG = 25
_NPG = 2000
_NE = 800000
_EPG = 32000
_LOG2 = 0.6931471805599453

_NW = 32
_BPW = _NE // _NW
_GC = 128
_NCH = (_BPW + _GC - 1) // _GC
_TAIL = _BPW - _GC

_EC = 4000
_ECH = _NE // _EC
_CPG = _EPG // _EC

_SC_CH = 640
_SC_NCH = _EPG // _SC_CH
_SC_GRP = _SC_CH // 16


def _sp2(x):
    return jax.nn.softplus(x) - _LOG2


def _sc_gather(table, idx_s, idx_d):
    mesh = plsc.VectorSubcoreMesh(core_axis_name="c", subcore_axis_name="s")

    @functools.partial(
        pl.kernel, mesh=mesh,
        compiler_params=pltpu.CompilerParams(use_tc_tiling_on_sc=False, needs_layout_passes=False),
        out_type=(jax.ShapeDtypeStruct((_NE, 32), jnp.float32),
                  jax.ShapeDtypeStruct((_NE, 32), jnp.float32)),
        scratch_types=[
            pltpu.VMEM((_NCH, _GC), jnp.int32),
            pltpu.VMEM((_NCH, _GC), jnp.int32),
            pltpu.VMEM((_GC, 32), jnp.float32),
            pltpu.VMEM((_GC, 32), jnp.float32),
            pltpu.VMEM((_GC, 32), jnp.float32),
            pltpu.VMEM((_GC, 32), jnp.float32),
            pltpu.SemaphoreType.DMA,
            pltpu.SemaphoreType.DMA,
            pltpu.SemaphoreType.DMA,
            pltpu.SemaphoreType.DMA,
        ],
    )
    def k(table_hbm, is_hbm, id_hbm, o1_hbm, o2_hbm, iv1, iv2,
          r1a, r2a, r1b, r2b, sa1, sa2, sb1, sb2):
        wid = lax.axis_index("s") * 2 + lax.axis_index("c")
        base = wid * _BPW
        pltpu.sync_copy(is_hbm.at[wid], iv1)
        pltpu.sync_copy(id_hbm.at[wid], iv2)

        def fire(c, r1, r2, s1, s2):
            pltpu.async_copy(table_hbm.at[iv1.at[c]], r1, s1)
            pltpu.async_copy(table_hbm.at[iv2.at[c]], r2, s2)

        def drain(i, r1, r2, s1, s2):
            off = base + jnp.minimum(i * _GC, _TAIL)
            pltpu.make_async_copy(o1_hbm.at[pl.ds(off, _GC)], r1, s1).wait()
            pltpu.make_async_copy(o2_hbm.at[pl.ds(off, _GC)], r2, s2).wait()
            pltpu.sync_copy(r1, o1_hbm.at[pl.ds(off, _GC)])
            pltpu.sync_copy(r2, o2_hbm.at[pl.ds(off, _GC)])

        fire(0, r1a, r2a, sa1, sa2)

        def body(t, carry):
            i0 = 2 * t
            fire(jnp.minimum(i0 + 1, _NCH - 1), r1b, r2b, sb1, sb2)
            drain(i0, r1a, r2a, sa1, sa2)
            fire(jnp.minimum(i0 + 2, _NCH - 1), r1a, r2a, sa1, sa2)
            drain(jnp.minimum(i0 + 1, _NCH - 1), r1b, r2b, sb1, sb2)
            return carry

        lax.fori_loop(0, _NCH // 2, body, 0)
        drain(_NCH - 1, r1a, r2a, sa1, sa2)

    return k(table, idx_s, idx_d)


def _sc_scatter(enew, dst, zrows, zcnt):
    mesh = plsc.VectorSubcoreMesh(core_axis_name="c", subcore_axis_name="s")

    @functools.partial(
        pl.kernel, mesh=mesh,
        compiler_params=pltpu.CompilerParams(use_tc_tiling_on_sc=False, needs_layout_passes=False),
        out_type=(jax.ShapeDtypeStruct((_NN, 32), jnp.float32),
                  jax.ShapeDtypeStruct((_NN, 16), jnp.float32)),
        scratch_types=[
            pltpu.VMEM((_SC_CH, 32), jnp.float32),
            pltpu.VMEM((_SC_CH,), jnp.int32),
            pltpu.VMEM((_NPG, 32), jnp.float32),
            pltpu.VMEM((_NPG, 16), jnp.float32),
        ],
    )
    def k(e_hbm, d_hbm, zr_hbm, zc_hbm, es_hbm, cn_hbm, ebuf, dbuf, acc, cnt):
        wid = lax.axis_index("s") * 2 + lax.axis_index("c")

        @pl.when(wid < _NG)
        def _():
            pltpu.sync_copy(zr_hbm, acc)
            pltpu.sync_copy(zc_hbm, cnt)
            ji = lax.iota(jnp.int32, 16)
            ones = jnp.ones((16,), jnp.float32)
            nbase = wid * _NPG
            ebase = wid * _EPG

            def chunk(i, carry):
                eb = ebase + i * _SC_CH
                pltpu.sync_copy(e_hbm.at[pl.ds(eb, _SC_CH)], ebuf)
                pltpu.sync_copy(d_hbm.at[pl.ds(eb, _SC_CH)], dbuf)

                def grp(j, c2):
                    dv = dbuf[pl.ds(j * 16, 16)]
                    loc = dv - nbase
                    rows = j * 16 + ji
                    plsc.addupdate_scatter(cnt, [loc, ji], ones)
                    for c in range(32):
                        cf = jnp.full((16,), c, jnp.int32)
                        vals = plsc.load_gather(ebuf, [rows, cf])
                        plsc.addupdate_scatter(acc, [loc, cf], vals)
                    return c2

                lax.fori_loop(0, _SC_GRP, grp, 0)
                return carry

            lax.fori_loop(0, _SC_NCH, chunk, 0)
            pltpu.sync_copy(acc, es_hbm.at[pl.ds(nbase, _NPG)])
            pltpu.sync_copy(cnt, cn_hbm.at[pl.ds(nbase, _NPG)])

    return k(enew, dst, zrows, zcnt)


def _dot(a, b):
    return jnp.dot(a, b, preferred_element_type=jnp.float32)


def _enc_edge_body(x_ref, w1, b1, w2, b2, o_ref):
    x = x_ref[...]
    g = jnp.exp(-(x * x) * 4.0)
    h = _sp2(g * w1[...] + b1[...])
    o_ref[...] = _sp2(_dot(h, w2[...]) + b2[...])


def _enc_edge(x, ps):
    (w1, b1), (w2, b2) = ps
    full = lambda s: pl.BlockSpec(s, lambda i: (0, 0))
    return pl.pallas_call(
        _enc_edge_body,
        grid=(_ECH,),
        in_specs=[pl.BlockSpec((_EC, 1), lambda i: (i, 0)),
                  full((1, 64)), full((1, 64)), full((64, 32)), full((1, 32))],
        out_specs=pl.BlockSpec((_EC, 32), lambda i: (i, 0)),
        out_shape=jax.ShapeDtypeStruct((_NE, 32), jnp.float32),
    )(x, w1.reshape(1, 64), b1.reshape(1, 64), w2, b2.reshape(1, 32))


def _enc_node_body(v_ref, emb, w1, b1, w2, b2, o_ref, f_ref):
    v = v_ref[...]
    ci = lax.broadcasted_iota(jnp.int32, (_NPG, 90), 1)
    sent = jnp.where((ci < 89) & (v > 0), ci, 2000)
    ntype = jnp.min(sent, axis=1, keepdims=True)
    ntype = jnp.where(ntype == 2000, 0, ntype)
    oh = (ntype == lax.broadcasted_iota(jnp.int32, (_NPG, 96), 1)).astype(jnp.float32)
    nf16 = _dot(oh, emb[...])
    h = _sp2(_dot(nf16, w1[...]) + b1[...])
    o_ref[...] = _sp2(_dot(h, w2[...]) + b2[...])
    flagv = jnp.sum(jnp.where(ci == 89, v, 0), axis=1, keepdims=True)
    li = lax.broadcasted_iota(jnp.int32, (_NPG, 1), 0)
    idxv = jnp.where(flagv != 0, li, 2000)
    fv = jnp.minimum(jnp.min(idxv), 20)
    f_ref[...] = jnp.full((1, 1, 128), fv.astype(jnp.float32))


def _enc_node(v, emb, ps):
    (w1, b1), (w2, b2) = ps
    embp = jnp.zeros((96, 16), jnp.float32).at[:89].set(emb)
    full = lambda s: pl.BlockSpec(s, lambda i: (0, 0))
    return pl.pallas_call(
        _enc_node_body,
        grid=(_NG,),
        in_specs=[pl.BlockSpec((_NPG, 90), lambda i: (i, 0)),
                  full((96, 16)), full((16, 64)), full((1, 64)),
                  full((64, 32)), full((1, 32))],
        out_specs=[pl.BlockSpec((_NPG, 32), lambda i: (i, 0)),
                   pl.BlockSpec((1, 1, 128), lambda i: (i, 0, 0))],
        out_shape=[jax.ShapeDtypeStruct((_NN, 32), jnp.float32),
                   jax.ShapeDtypeStruct((_NG, 1, 128), jnp.float32)],
    )(v, embp, w1, b1.reshape(1, 64), w2, b2.reshape(1, 32))


def _enc_state_body(sfeat, focus, semb, w1, b1, w2, b2, o_ref):
    fv = focus[...][:, 0:1].astype(jnp.int32)
    oh = (fv == lax.broadcasted_iota(jnp.int32, (32, 24), 1)).astype(jnp.float32)
    ff = _dot(oh, semb[...])
    x = jnp.concatenate([sfeat[...], ff], axis=1)
    h = _sp2(_dot(x, w1[...]) + b1[...])
    o_ref[...] = _sp2(_dot(h, w2[...]) + b2[...])


def _enc_state(sfeat, focus, semb, ps):
    (w1, b1), (w2, b2) = ps
    sembp = jnp.zeros((24, 8), jnp.float32).at[:21].set(semb)
    sfp = jnp.zeros((32, 8), jnp.float32).at[:_NG].set(sfeat)
    fp = jnp.zeros((32, 128), jnp.float32).at[:_NG].set(focus)
    return pl.pallas_call(
        _enc_state_body,
        out_shape=jax.ShapeDtypeStruct((32, 32), jnp.float32),
    )(sfp, fp, sembp, w1, b1.reshape(1, 64), w2, b2.reshape(1, 32))


def _dense32_body(x_ref, w, b, o_ref):
    o_ref[...] = _sp2(_dot(x_ref[...], w[...]) + b[...])


def _dense_table(x, ps):
    (w, b), = ps
    full = lambda s: pl.BlockSpec(s, lambda i: (0, 0))
    return pl.pallas_call(
        _dense32_body,
        grid=(_NG,),
        in_specs=[pl.BlockSpec((_NPG, 32), lambda i: (i, 0)),
                  full((32, 32)), full((1, 32))],
        out_specs=pl.BlockSpec((_NPG, 32), lambda i: (i, 0)),
        out_shape=jax.ShapeDtypeStruct((_NN, 32), jnp.float32),
    )(x, w, b.reshape(1, 32))


def _dense_state(x, ps):
    (w, b), = ps
    return pl.pallas_call(
        _dense32_body,
        out_shape=jax.ShapeDtypeStruct((32, 32), jnp.float32),
    )(x, w, b.reshape(1, 32))


def _onehot_row(ref, g, rows):
    sel = (lax.broadcasted_iota(jnp.int32, (rows, 1), 0) == g).astype(jnp.float32)
    return _dot(sel.reshape(1, rows), ref[...])


def _edge_block_body(has_dense):
    def body(ep_ref, gs_ref, gd_ref, sd_ref,
             wd, bd, w1s, w1d, w1e, w1u, b1, w2, b2, w3, b3,
             en_ref, eo_ref, ps_ref):
        g = pl.program_id(0) // _CPG
        sg = _onehot_row(sd_ref, g, 32)
        ep = ep_ref[...]
        ed = _sp2(_dot(ep, wd[...]) + bd[...]) if has_dense else ep
        h1 = _sp2(_dot(gs_ref[...], w1s[...]) + _dot(gd_ref[...], w1d[...])
                  + _dot(ed, w1e[...]) + _dot(sg, w1u[...]) + b1[...])
        h2 = _sp2(_dot(h1, w2[...]) + b2[...])
        en = _sp2(_dot(h2, w3[...]) + b3[...])
        eo = en + ep
        en_ref[...] = en
        eo_ref[...] = eo
        s1 = jnp.sum(en, axis=0, keepdims=True)
        s2 = jnp.sum(eo, axis=0, keepdims=True)
        ps_ref[...] = jnp.concatenate(
            [s1, s2, jnp.zeros((1, 64), jnp.float32)], axis=1).reshape(1, 1, 128)
    return body


def _edge_block(ep, gs, gd, sd, bp, has_dense):
    (w1, b1), (w2, b2), (w3, b3) = bp['edge_func']
    if has_dense:
        (wd, bd), = bp['edge_dense']
    else:
        wd = jnp.zeros((32, 32), jnp.float32)
        bd = jnp.zeros((32,), jnp.float32)
    full = lambda s: pl.BlockSpec(s, lambda i: (0, 0))
    eb = lambda: pl.BlockSpec((_EC, 32), lambda i: (i, 0))
    return pl.pallas_call(
        _edge_block_body(has_dense),
        grid=(_ECH,),
        in_specs=[eb(), eb(), eb(), full((32, 32)),
                  full((32, 32)), full((1, 32)),
                  full((32, 64)), full((32, 64)), full((32, 64)), full((32, 64)),
                  full((1, 64)), full((64, 64)), full((1, 64)),
                  full((64, 32)), full((1, 32))],
        out_specs=[eb(), eb(), pl.BlockSpec((1, 1, 128), lambda i: (i, 0, 0))],
        out_shape=[jax.ShapeDtypeStruct((_NE, 32), jnp.float32),
                   jax.ShapeDtypeStruct((_NE, 32), jnp.float32),
                   jax.ShapeDtypeStruct((_ECH, 1, 128), jnp.float32)],
    )(ep, gs, gd, sd, wd, bd.reshape(1, 32),
      w1[0:32], w1[32:64], w1[64:96], w1[96:128], b1.reshape(1, 64),
      w2, b2.reshape(1, 64), w3, b3.reshape(1, 32))


def _node_block_body(np_ref, tb_ref, es_ref, ct_ref, sd_ref,
                     w1n, w1v, w1u, b1, w2, b2, w3, b3,
                     no_ref, ps_ref):
    g = pl.program_id(0)
    sg = _onehot_row(sd_ref, g, 32)
    ve = es_ref[...] / jnp.maximum(jnp.sum(ct_ref[...], axis=1, keepdims=True), 1.0)
    h1 = _sp2(_dot(tb_ref[...], w1n[...]) + _dot(ve, w1v[...])
              + _dot(sg, w1u[...]) + b1[...])
    h2 = _sp2(_dot(h1, w2[...]) + b2[...])
    nn = _sp2(_dot(h2, w3[...]) + b3[...])
    no = nn + np_ref[...]
    no_ref[...] = no
    s1 = jnp.sum(nn, axis=0, keepdims=True)
    s2 = jnp.sum(no, axis=0, keepdims=True)
    ps_ref[...] = jnp.concatenate(
        [s1, s2, jnp.zeros((1, 64), jnp.float32)], axis=1).reshape(1, 1, 128)


def _node_block(nprev, table, esum, cnt, sd, bp):
    (w1, b1), (w2, b2), (w3, b3) = bp['node_func']
    full = lambda s: pl.BlockSpec(s, lambda i: (0, 0))
    nb = lambda: pl.BlockSpec((_NPG, 32), lambda i: (i, 0))
    return pl.pallas_call(
        _node_block_body,
        grid=(_NG,),
        in_specs=[nb(), nb(), nb(), pl.BlockSpec((_NPG, 16), lambda i: (i, 0)),
                  full((32, 32)),
                  full((32, 64)), full((32, 64)), full((32, 64)), full((1, 64)),
                  full((64, 64)), full((1, 64)), full((64, 32)), full((1, 32))],
        out_specs=[nb(), pl.BlockSpec((1, 1, 128), lambda i: (i, 0, 0))],
        out_shape=[jax.ShapeDtypeStruct((_NN, 32), jnp.float32),
                   jax.ShapeDtypeStruct((_NG, 1, 128), jnp.float32)],
    )(nprev, table, esum, cnt, sd,
      w1[0:32], w1[32:64], w1[64:96], b1.reshape(1, 64),
      w2, b2.reshape(1, 64), w3, b3.reshape(1, 32))


def _state_func_body(sp_ref, sd_ref, ue_ref, uv_ref, w1s, w1e, w1v, b1, w2, b2, w3, b3, o_ref):
    h1 = _sp2(_dot(sd_ref[...], w1s[...]) + _dot(ue_ref[...], w1e[...])
              + _dot(uv_ref[...], w1v[...]) + b1[...])
    h2 = _sp2(_dot(h1, w2[...]) + b2[...])
    o_ref[...] = _sp2(_dot(h2, w3[...]) + b3[...]) + sp_ref[...]


def _state_func(sprev, sd, ue, uv, bp):
    (w1, b1), (w2, b2), (w3, b3) = bp['state_func']
    return pl.pallas_call(
        _state_func_body,
        out_shape=jax.ShapeDtypeStruct((32, 32), jnp.float32),
    )(sprev, sd, ue, uv,
      w1[0:32], w1[32:64], w1[64:96], b1.reshape(1, 64),
      w2, b2.reshape(1, 64), w3, b3.reshape(1, 32))


def _final_body(nm_ref, em_ref, s3_ref, w1, b1, w2, b2, w3, b3, o_ref):
    z = jnp.zeros((32, 32), jnp.float32)
    x = jnp.concatenate([z, nm_ref[...], z, em_ref[...], s3_ref[...],
                         jnp.zeros((32, 96), jnp.float32)], axis=1)
    h1 = _sp2(_dot(x, w1[...]) + b1[...])
    h2 = _sp2(_dot(h1, w2[...]) + b2[...])
    o_ref[...] = _dot(h2, w3[...]) + b3[...]


def _final(nmean, emean, s3, ps):
    (w1, b1), (w2, b2), (w3, b3) = ps
    w1p = jnp.zeros((256, 32), jnp.float32).at[:160].set(w1)
    return pl.pallas_call(
        _final_body,
        out_shape=jax.ShapeDtypeStruct((32, 88), jnp.float32),
    )(nmean, emean, s3, w1p, b1.reshape(1, 32),
      w2, b2.reshape(1, 16), w3, b3.reshape(1, 88))


def _mk_gidx(idx):
    x = idx.reshape(_NW, _BPW)
    main = x[:, :_GC * (_NCH - 1)].reshape(_NW, _NCH - 1, _GC)
    tail = x[:, _TAIL:][:, None, :]
    return jnp.concatenate([main, tail], axis=1)


def kernel(edge_index, edge_feat, node_feat, state_feat, params):
    p = params
    src = edge_index[0].astype(jnp.int32)
    dst = edge_index[1].astype(jnp.int32)
    nfi = node_feat.astype(jnp.int32)

    ef = _enc_edge(edge_feat.reshape(_NE, 1), p['edge_enc'])
    nf, focus = _enc_node(nfi, p['node_embed'], p['node_enc'])
    sf = _enc_state(state_feat, focus[:, 0, :], p['state_embed'], p['state_enc'])

    gidx_s = _mk_gidx(src)
    gidx_d = _mk_gidx(dst)
    zrows = jnp.zeros((_NPG, 32), jnp.float32)
    zcnt = jnp.zeros((_NPG, 16), jnp.float32)

    epart3 = None
    npart3 = None
    for b, bp in enumerate(p['blocks']):
        if b > 0:
            table = _dense_table(nf, bp['node_dense'])
            sd = _dense_state(sf, bp['state_dense'])
        else:
            table, sd = nf, sf
        gs, gd = _sc_gather(table, gidx_s, gidx_d)
        enew, eout, epart = _edge_block(ef, gs, gd, sd, bp, has_dense=b > 0)
        esum, cnt = _sc_scatter(enew, dst, zrows, zcnt)
        nout, npart = _node_block(nf, table, esum, cnt, sd, bp)
        eg = epart.reshape(_NG, _CPG, 128).sum(axis=1)
        ng = npart[:, 0, :]
        ue = jnp.zeros((32, 32), jnp.float32).at[:_NG].set(eg[:, 0:32] / _EPG)
        uv = jnp.zeros((32, 32), jnp.float32).at[:_NG].set(ng[:, 0:32] / _NPG)
        sf = _state_func(sf, sd, ue, uv, bp)
        ef, nf = eout, nout
        epart3, npart3 = eg, ng

    nmean = jnp.zeros((32, 32), jnp.float32).at[:_NG].set(npart3[:, 32:64] / _NPG)
    emean = jnp.zeros((32, 32), jnp.float32).at[:_NG].set(epart3[:, 32:64] / _EPG)
    out = _final(nmean, emean, sf, p['out_proj'])
    return out[:_NG]
